# baseline (speedup 1.0000x reference)
import functools

import jax
import jax.numpy as jnp
from jax import lax
from jax.experimental import pallas as pl
from jax.experimental.pallas import tpu as pltpu
from jax.experimental.pallas import tpu_sc as plsc

N_PAD = 10240
EB = 128
D = 64
NC, NS = 2, 16
NW = NC * NS
NPB = 80
EPW = NPB * EB
E_PAD = NW * EPW
RPS = N_PAD // NS
G = 4
GEB = G * EB
NGRP = NPB // G
NBUF = 8
N_GRAPHS = 64


def _sc_mesh():
    return plsc.VectorSubcoreMesh(core_axis_name="c", subcore_axis_name="s")


_DEG_LAG = 8


def _deg_body(dst_hbm, ones_hbm, zeros_hbm, out_hbm, idx_d, ones_v, zsem, sem,
              acc_sh):
    c = lax.axis_index("c")
    s = lax.axis_index("s")
    wid = c * NS + s
    r0 = s * RPS

    pltpu.async_copy(dst_hbm.at[wid], idx_d, sem)
    pltpu.async_copy(zeros_hbm, acc_sh.at[pl.ds(r0, RPS)], zsem)
    pltpu.sync_copy(ones_hbm, ones_v)
    pltpu.make_async_copy(zeros_hbm, acc_sh.at[pl.ds(r0, RPS)], zsem).wait()
    pltpu.make_async_copy(dst_hbm.at[wid], idx_d, sem).wait()
    plsc.subcore_barrier()

    def eloop(i, _):
        pltpu.async_copy(ones_v, acc_sh.at[idx_d.at[i]], sem, add=True)

        @pl.when(i >= _DEG_LAG)
        def _drain():
            pltpu.make_async_copy(ones_hbm, ones_v, sem).wait()

        return _

    lax.fori_loop(0, NPB, eloop, None)
    for _ in range(_DEG_LAG):
        pltpu.make_async_copy(ones_hbm, ones_v, sem).wait()
    plsc.subcore_barrier()

    pltpu.async_copy(acc_sh.at[pl.ds(r0, RPS)], out_hbm.at[c, pl.ds(r0, RPS)],
                     zsem)
    pltpu.make_async_copy(acc_sh.at[pl.ds(r0, RPS)],
                          out_hbm.at[c, pl.ds(r0, RPS)], zsem).wait()


_deg_kernel = functools.partial(
    pl.kernel,
    out_type=jax.ShapeDtypeStruct((NC, N_PAD, 8), jnp.float32),
    mesh=_sc_mesh(),
    compiler_params=pltpu.CompilerParams(use_tc_tiling_on_sc=False),
    scratch_types=[
        pltpu.VMEM((NPB, EB), jnp.int32),
        pltpu.VMEM((EB, 8), jnp.float32),
        pltpu.SemaphoreType.DMA,
        pltpu.SemaphoreType.DMA,
        pltpu.VMEM_SHARED((N_PAD, 8), jnp.float32),
    ],
)(_deg_body)


def _agg_body(y_hbm, src_hbm, dst_hbm, zeros_hbm, out_hbm, idx_s, idx_d, rows,
              gsem, ssem, acc_sh):
    c = lax.axis_index("c")
    s = lax.axis_index("s")
    wid = c * NS + s
    r0 = s * RPS

    pltpu.async_copy(src_hbm.at[wid], idx_s, gsem.at[0])
    pltpu.async_copy(dst_hbm.at[wid], idx_d, gsem.at[1])
    pltpu.async_copy(zeros_hbm, acc_sh.at[pl.ds(r0, RPS)], ssem.at[0])
    pltpu.make_async_copy(zeros_hbm, acc_sh.at[pl.ds(r0, RPS)],
                          ssem.at[0]).wait()
    pltpu.make_async_copy(src_hbm.at[wid], idx_s, gsem.at[0]).wait()
    pltpu.make_async_copy(dst_hbm.at[wid], idx_d, gsem.at[1]).wait()
    plsc.subcore_barrier()

    for p in range(2):
        pltpu.async_copy(y_hbm.at[idx_s.at[p]], rows.at[p], gsem.at[p])

    def eloop(g, _):
        p = lax.rem(g, 2)
        pltpu.make_async_copy(y_hbm.at[pl.ds(0, GEB)], rows.at[p],
                              gsem.at[p]).wait()
        pltpu.async_copy(rows.at[p], acc_sh.at[idx_d.at[g]], ssem.at[p],
                         add=True)
        pltpu.make_async_copy(y_hbm.at[pl.ds(0, GEB)], rows.at[p],
                              ssem.at[p]).wait()

        @pl.when(g + 2 < NGRP)
        def _next():
            pltpu.async_copy(y_hbm.at[idx_s.at[g + 2]], rows.at[p],
                             gsem.at[p])

        return _

    lax.fori_loop(0, NGRP, eloop, None)
    plsc.subcore_barrier()

    pltpu.async_copy(acc_sh.at[pl.ds(r0, RPS)], out_hbm.at[c, pl.ds(r0, RPS)],
                     gsem.at[0])
    pltpu.make_async_copy(acc_sh.at[pl.ds(r0, RPS)],
                          out_hbm.at[c, pl.ds(r0, RPS)], gsem.at[0]).wait()


_agg64 = functools.partial(
    pl.kernel,
    out_type=jax.ShapeDtypeStruct((NC, N_PAD, D), jnp.float32),
    mesh=_sc_mesh(),
    compiler_params=pltpu.CompilerParams(use_tc_tiling_on_sc=False),
    scratch_types=[
        pltpu.VMEM((NGRP, GEB), jnp.int32),
        pltpu.VMEM((NGRP, GEB), jnp.int32),
        pltpu.VMEM((2, GEB, D), jnp.float32),
        pltpu.SemaphoreType.DMA((2,)),
        pltpu.SemaphoreType.DMA((2,)),
        pltpu.VMEM_SHARED((N_PAD, D), jnp.float32),
    ],
)(_agg_body)


def _t1a_body(x_ref, w_ref, xw_ref):
    xw_ref[...] = jnp.dot(x_ref[...], w_ref[...],
                          preferred_element_type=jnp.float32)


def _t1b_body(xw_ref, deg_ref, y_ref, dinv_ref):
    deg = deg_ref[0, :, 0:1] + deg_ref[1, :, 0:1] + 1.0
    dinv = lax.rsqrt(deg)
    y_ref[...] = dinv * xw_ref[...]
    dinv_ref[...] = jnp.broadcast_to(dinv, (N_PAD, 8))


def _t2_body(agg_ref, y_ref, dinv_ref, b_ref, wa_ref, wb_ref, ya_ref, yb_ref):
    dinv = dinv_ref[:, 0:1]
    h = jnp.maximum(dinv * (agg_ref[0] + agg_ref[1] + y_ref[...]) + b_ref[...],
                    0.0)
    ya_ref[...] = dinv * jnp.dot(h, wa_ref[...],
                                 preferred_element_type=jnp.float32)
    yb_ref[...] = dinv * jnp.dot(h, wb_ref[...],
                                 preferred_element_type=jnp.float32)


def _t3_body(agga_ref, aggb_ref, ya_ref, yb_ref, dinv_ref, ba_ref, bb_ref,
             wa_ref, wb_ref, out_ref):
    dinv = dinv_ref[:, 0:1]
    ha = jnp.maximum(
        dinv * (agga_ref[0] + agga_ref[1] + ya_ref[...]) + ba_ref[...], 0.0)
    hb = jnp.maximum(
        dinv * (aggb_ref[0] + aggb_ref[1] + yb_ref[...]) + bb_ref[...], 0.0)
    out_ref[...] = dinv * (
        jnp.dot(ha, wa_ref[...], preferred_element_type=jnp.float32)
        + jnp.dot(hb, wb_ref[...], preferred_element_type=jnp.float32))


def _t4_body(agg_ref, y_ref, dinv_ref, b_ref, batch_ref, wlin_ref, blin_ref,
             out_ref):
    dinv = dinv_ref[:, 0:1]
    h = jnp.maximum(dinv * (agg_ref[0] + agg_ref[1] + y_ref[...]) + b_ref[...],
                    0.0)
    gids = lax.broadcasted_iota(jnp.int32, (N_GRAPHS, N_PAD), 0)
    onehot = (batch_ref[...] == gids).astype(jnp.float32)
    cnts = jnp.sum(onehot, axis=1, keepdims=True)
    sums = jnp.dot(onehot, h, preferred_element_type=jnp.float32)
    pooled = sums / jnp.maximum(cnts, 1.0)
    out_ref[...] = jnp.dot(pooled, wlin_ref[...],
                           preferred_element_type=jnp.float32) + blin_ref[...]


def kernel(x, edge_index, batch, W1, b1, W2, b2, W3, b3, Wlin, blin):
    n = x.shape[0]
    e = edge_index.shape[1]

    fill = n + (jnp.arange(E_PAD - e, dtype=jnp.int32) % (N_PAD - 8 - n))
    dst = jnp.concatenate([edge_index[1], fill]).reshape(NW, NPB, EB)
    src4 = jnp.concatenate([edge_index[0], fill]).reshape(NW, NGRP, GEB)
    dst4 = dst.reshape(NW, NGRP, GEB)
    x_p = jnp.pad(x, ((0, N_PAD - n), (0, 0)))
    batch_p = jnp.pad(batch, (0, N_PAD - n),
                      constant_values=N_GRAPHS).reshape(1, N_PAD)

    ones8 = jnp.ones((EB, 8), jnp.float32)
    zeros8 = jnp.zeros((RPS, 8), jnp.float32)
    zeros64 = jnp.zeros((RPS, D), jnp.float32)

    degraw = _deg_kernel(dst, ones8, zeros8)

    xw = pl.pallas_call(
        _t1a_body,
        out_shape=jax.ShapeDtypeStruct((N_PAD, 64), jnp.float32),
    )(x_p, W1)

    y1, dinv8 = pl.pallas_call(
        _t1b_body,
        out_shape=[
            jax.ShapeDtypeStruct((N_PAD, 64), jnp.float32),
            jax.ShapeDtypeStruct((N_PAD, 8), jnp.float32),
        ],
    )(xw, degraw)

    agg1 = _agg64(y1, src4, dst4, zeros64)

    y2a, y2b = pl.pallas_call(
        _t2_body,
        out_shape=[
            jax.ShapeDtypeStruct((N_PAD, 64), jnp.float32),
            jax.ShapeDtypeStruct((N_PAD, 64), jnp.float32),
        ],
    )(agg1, y1, dinv8, b1.reshape(1, 64), W2[:, :64], W2[:, 64:])

    agg2a = _agg64(y2a, src4, dst4, zeros64)
    agg2b = _agg64(y2b, src4, dst4, zeros64)

    y3 = pl.pallas_call(
        _t3_body,
        out_shape=jax.ShapeDtypeStruct((N_PAD, 64), jnp.float32),
    )(agg2a, agg2b, y2a, y2b, dinv8, b2[:64].reshape(1, 64),
      b2[64:].reshape(1, 64), W3[:64], W3[64:])

    agg3 = _agg64(y3, src4, dst4, zeros64)

    out = pl.pallas_call(
        _t4_body,
        out_shape=jax.ShapeDtypeStruct((N_GRAPHS, 1), jnp.float32),
    )(agg3, y3, dinv8, b3.reshape(1, 64), batch_p, Wlin, blin.reshape(1, 1))

    return out


# speedup vs baseline: 1.0459x; 1.0459x over previous
import functools

import jax
import jax.numpy as jnp
from jax import lax
from jax.experimental import pallas as pl
from jax.experimental.pallas import tpu as pltpu
from jax.experimental.pallas import tpu_sc as plsc

N_PAD = 10240
EB = 128
D = 64
NC, NS = 2, 16
NW = NC * NS
NPB = 80
EPW = NPB * EB
E_PAD = NW * EPW
RPS = N_PAD // NS
GEB = 512
NGRP = EPW // GEB
MGEB = 256
MNGRP = 2 * EPW // MGEB
NBUF = 8
N_GRAPHS = 64


def _sc_mesh():
    return plsc.VectorSubcoreMesh(core_axis_name="c", subcore_axis_name="s")


_DEG_LAG = 8


def _deg_body(dst_hbm, ones_hbm, zeros_hbm, out_hbm, idx_d, ones_v, zsem, sem,
              acc_sh):
    c = lax.axis_index("c")
    s = lax.axis_index("s")
    wid = c * NS + s
    r0 = s * RPS

    pltpu.async_copy(dst_hbm.at[wid], idx_d, sem)
    pltpu.async_copy(zeros_hbm, acc_sh.at[pl.ds(r0, RPS)], zsem)
    pltpu.sync_copy(ones_hbm, ones_v)
    pltpu.make_async_copy(zeros_hbm, acc_sh.at[pl.ds(r0, RPS)], zsem).wait()
    pltpu.make_async_copy(dst_hbm.at[wid], idx_d, sem).wait()
    plsc.subcore_barrier()

    def eloop(i, _):
        pltpu.async_copy(ones_v, acc_sh.at[idx_d.at[i]], sem, add=True)

        @pl.when(i >= _DEG_LAG)
        def _drain():
            pltpu.make_async_copy(ones_hbm, ones_v, sem).wait()

        return _

    lax.fori_loop(0, NPB, eloop, None)
    for _ in range(_DEG_LAG):
        pltpu.make_async_copy(ones_hbm, ones_v, sem).wait()
    plsc.subcore_barrier()

    pltpu.async_copy(acc_sh.at[pl.ds(r0, RPS)], out_hbm.at[c, pl.ds(r0, RPS)],
                     zsem)
    pltpu.make_async_copy(acc_sh.at[pl.ds(r0, RPS)],
                          out_hbm.at[c, pl.ds(r0, RPS)], zsem).wait()


_deg_kernel = functools.partial(
    pl.kernel,
    out_type=jax.ShapeDtypeStruct((NC, N_PAD, 8), jnp.float32),
    mesh=_sc_mesh(),
    compiler_params=pltpu.CompilerParams(use_tc_tiling_on_sc=False),
    scratch_types=[
        pltpu.VMEM((NPB, EB), jnp.int32),
        pltpu.VMEM((EB, 8), jnp.float32),
        pltpu.SemaphoreType.DMA,
        pltpu.SemaphoreType.DMA,
        pltpu.VMEM_SHARED((N_PAD, 8), jnp.float32),
    ],
)(_deg_body)


def _agg_body(y_hbm, src_hbm, dst_hbm, zeros_hbm, out_hbm, idx_s, idx_d, rows,
              gsem, ssem, acc_sh):
    c = lax.axis_index("c")
    s = lax.axis_index("s")
    wid = c * NS + s
    r0 = s * RPS
    ngrp, geb = idx_s.shape

    pltpu.async_copy(src_hbm.at[wid], idx_s, gsem.at[0])
    pltpu.async_copy(dst_hbm.at[wid], idx_d, gsem.at[1])
    pltpu.async_copy(zeros_hbm, acc_sh.at[pl.ds(r0, RPS)], ssem.at[0])
    pltpu.make_async_copy(zeros_hbm, acc_sh.at[pl.ds(r0, RPS)],
                          ssem.at[0]).wait()
    pltpu.make_async_copy(src_hbm.at[wid], idx_s, gsem.at[0]).wait()
    pltpu.make_async_copy(dst_hbm.at[wid], idx_d, gsem.at[1]).wait()
    plsc.subcore_barrier()

    for p in range(2):
        pltpu.async_copy(y_hbm.at[idx_s.at[p]], rows.at[p], gsem.at[p])

    def eloop(g, _):
        p = lax.rem(g, 2)
        pltpu.make_async_copy(y_hbm.at[pl.ds(0, geb)], rows.at[p],
                              gsem.at[p]).wait()
        pltpu.async_copy(rows.at[p], acc_sh.at[idx_d.at[g]], ssem.at[p],
                         add=True)
        pltpu.make_async_copy(y_hbm.at[pl.ds(0, geb)], rows.at[p],
                              ssem.at[p]).wait()

        @pl.when(g + 2 < ngrp)
        def _next():
            pltpu.async_copy(y_hbm.at[idx_s.at[g + 2]], rows.at[p],
                             gsem.at[p])

        return _

    lax.fori_loop(0, ngrp, eloop, None)
    plsc.subcore_barrier()

    pltpu.async_copy(acc_sh.at[pl.ds(r0, RPS)], out_hbm.at[c, pl.ds(r0, RPS)],
                     gsem.at[0])
    pltpu.make_async_copy(acc_sh.at[pl.ds(r0, RPS)],
                          out_hbm.at[c, pl.ds(r0, RPS)], gsem.at[0]).wait()


_agg64 = functools.partial(
    pl.kernel,
    out_type=jax.ShapeDtypeStruct((NC, N_PAD, D), jnp.float32),
    mesh=_sc_mesh(),
    compiler_params=pltpu.CompilerParams(use_tc_tiling_on_sc=False),
    scratch_types=[
        pltpu.VMEM((NGRP, GEB), jnp.int32),
        pltpu.VMEM((NGRP, GEB), jnp.int32),
        pltpu.VMEM((2, GEB, D), jnp.float32),
        pltpu.SemaphoreType.DMA((2,)),
        pltpu.SemaphoreType.DMA((2,)),
        pltpu.VMEM_SHARED((N_PAD, D), jnp.float32),
    ],
)(_agg_body)

_agg64m = functools.partial(
    pl.kernel,
    out_type=jax.ShapeDtypeStruct((NC, N_PAD, D), jnp.float32),
    mesh=_sc_mesh(),
    compiler_params=pltpu.CompilerParams(use_tc_tiling_on_sc=False),
    scratch_types=[
        pltpu.VMEM((MNGRP, MGEB), jnp.int32),
        pltpu.VMEM((MNGRP, MGEB), jnp.int32),
        pltpu.VMEM((2, MGEB, D), jnp.float32),
        pltpu.SemaphoreType.DMA((2,)),
        pltpu.SemaphoreType.DMA((2,)),
        pltpu.VMEM_SHARED((N_PAD, D), jnp.float32),
    ],
)(_agg_body)


def _t1_body(x_ref, w_ref, deg_ref, y_ref, dinv_ref):
    deg = deg_ref[0, :, 0:1] + deg_ref[1, :, 0:1] + 1.0
    dinv = lax.rsqrt(deg)
    y_ref[...] = dinv * jnp.dot(x_ref[...], w_ref[...],
                                preferred_element_type=jnp.float32)
    dinv_ref[...] = jnp.broadcast_to(dinv, (N_PAD, 8))


def _t2_body(agg_ref, y_ref, dinv_ref, b_ref, wa_ref, wb_ref, y2_ref):
    dinv = dinv_ref[:, 0:1]
    h = jnp.maximum(dinv * (agg_ref[0] + agg_ref[1] + y_ref[...]) + b_ref[...],
                    0.0)
    y2_ref[0] = dinv * jnp.dot(h, wa_ref[...],
                               preferred_element_type=jnp.float32)
    y2_ref[1] = dinv * jnp.dot(h, wb_ref[...],
                               preferred_element_type=jnp.float32)


def _t3_body(agg_ref, y2_ref, dinv_ref, ba_ref, bb_ref, wa_ref, wb_ref,
             out_ref):
    dinv = dinv_ref[:, 0:1]
    ha = jnp.maximum(dinv * (agg_ref[0] + y2_ref[0]) + ba_ref[...], 0.0)
    hb = jnp.maximum(dinv * (agg_ref[1] + y2_ref[1]) + bb_ref[...], 0.0)
    out_ref[...] = dinv * (
        jnp.dot(ha, wa_ref[...], preferred_element_type=jnp.float32)
        + jnp.dot(hb, wb_ref[...], preferred_element_type=jnp.float32))


def _t4_body(agg_ref, y_ref, dinv_ref, b_ref, batch_ref, wlin_ref, blin_ref,
             out_ref):
    dinv = dinv_ref[:, 0:1]
    h = jnp.maximum(dinv * (agg_ref[0] + agg_ref[1] + y_ref[...]) + b_ref[...],
                    0.0)
    gids = lax.broadcasted_iota(jnp.int32, (N_GRAPHS, N_PAD), 0)
    onehot = (batch_ref[...] == gids).astype(jnp.float32)
    cnts = jnp.sum(onehot, axis=1, keepdims=True)
    sums = jnp.dot(onehot, h, preferred_element_type=jnp.float32)
    pooled = sums / jnp.maximum(cnts, 1.0)
    out_ref[...] = jnp.dot(pooled, wlin_ref[...],
                           preferred_element_type=jnp.float32) + blin_ref[...]


def kernel(x, edge_index, batch, W1, b1, W2, b2, W3, b3, Wlin, blin):
    n = x.shape[0]
    e = edge_index.shape[1]

    fill = n + (jnp.arange(E_PAD - e, dtype=jnp.int32) % (N_PAD - 8 - n))
    srcfull = jnp.concatenate([edge_index[0], fill])
    dstfull = jnp.concatenate([edge_index[1], fill])
    dst = dstfull.reshape(NW, NPB, EB)
    src4 = srcfull.reshape(NW, NGRP, GEB)
    dst4 = dst.reshape(NW, NGRP, GEB)

    src_sub = srcfull.reshape(NS, 2 * EPW)
    dst_sub = dstfull.reshape(NS, 2 * EPW)
    src_m = jnp.stack([src_sub, src_sub + N_PAD]).reshape(NW, MNGRP, MGEB)
    dst_m = jnp.stack([dst_sub, dst_sub]).reshape(NW, MNGRP, MGEB)
    x_p = jnp.pad(x, ((0, N_PAD - n), (0, 0)))
    batch_p = jnp.pad(batch, (0, N_PAD - n),
                      constant_values=N_GRAPHS).reshape(1, N_PAD)

    ones8 = jnp.ones((EB, 8), jnp.float32)
    zeros8 = jnp.zeros((RPS, 8), jnp.float32)
    zeros64 = jnp.zeros((RPS, D), jnp.float32)

    degraw = _deg_kernel(dst, ones8, zeros8)

    y1, dinv8 = pl.pallas_call(
        _t1_body,
        out_shape=[
            jax.ShapeDtypeStruct((N_PAD, 64), jnp.float32),
            jax.ShapeDtypeStruct((N_PAD, 8), jnp.float32),
        ],
    )(x_p, W1, degraw)

    agg1 = _agg64(y1, src4, dst4, zeros64)

    y2 = pl.pallas_call(
        _t2_body,
        out_shape=jax.ShapeDtypeStruct((2, N_PAD, 64), jnp.float32),
    )(agg1, y1, dinv8, b1.reshape(1, 64), W2[:, :64], W2[:, 64:])

    agg2 = _agg64m(y2.reshape(2 * N_PAD, 64), src_m, dst_m, zeros64)

    y3 = pl.pallas_call(
        _t3_body,
        out_shape=jax.ShapeDtypeStruct((N_PAD, 64), jnp.float32),
    )(agg2, y2, dinv8, b2[:64].reshape(1, 64),
      b2[64:].reshape(1, 64), W3[:64], W3[64:])

    agg3 = _agg64(y3, src4, dst4, zeros64)

    out = pl.pallas_call(
        _t4_body,
        out_shape=jax.ShapeDtypeStruct((N_GRAPHS, 1), jnp.float32),
    )(agg3, y3, dinv8, b3.reshape(1, 64), batch_p, Wlin, blin.reshape(1, 1))

    return out


# speedup vs baseline: 1.0694x; 1.0225x over previous
import functools

import jax
import jax.numpy as jnp
from jax import lax
from jax.experimental import pallas as pl
from jax.experimental.pallas import tpu as pltpu
from jax.experimental.pallas import tpu_sc as plsc

N_PAD = 10240
EB = 128
D = 64
NC, NS = 2, 16
NW = NC * NS
NPB = 80
EPW = NPB * EB
E_PAD = NW * EPW
RPS = N_PAD // NS
GEB = 256
NGRP = EPW // GEB
MGEB = 256
MNGRP = 2 * EPW // MGEB
NBUF = 8
N_GRAPHS = 64


def _sc_mesh():
    return plsc.VectorSubcoreMesh(core_axis_name="c", subcore_axis_name="s")


_DEG_LAG = 8


def _deg_body(dst_hbm, ones_hbm, zeros_hbm, out_hbm, idx_d, ones_v, zsem, sem,
              acc_sh):
    c = lax.axis_index("c")
    s = lax.axis_index("s")
    wid = c * NS + s
    r0 = s * RPS

    pltpu.async_copy(dst_hbm.at[wid], idx_d, sem)
    pltpu.async_copy(zeros_hbm, acc_sh.at[pl.ds(r0, RPS)], zsem)
    pltpu.sync_copy(ones_hbm, ones_v)
    pltpu.make_async_copy(zeros_hbm, acc_sh.at[pl.ds(r0, RPS)], zsem).wait()
    pltpu.make_async_copy(dst_hbm.at[wid], idx_d, sem).wait()
    plsc.subcore_barrier()

    def eloop(i, _):
        pltpu.async_copy(ones_v, acc_sh.at[idx_d.at[i]], sem, add=True)

        @pl.when(i >= _DEG_LAG)
        def _drain():
            pltpu.make_async_copy(ones_hbm, ones_v, sem).wait()

        return _

    lax.fori_loop(0, NPB, eloop, None)
    for _ in range(_DEG_LAG):
        pltpu.make_async_copy(ones_hbm, ones_v, sem).wait()
    plsc.subcore_barrier()

    pltpu.async_copy(acc_sh.at[pl.ds(r0, RPS)], out_hbm.at[c, pl.ds(r0, RPS)],
                     zsem)
    pltpu.make_async_copy(acc_sh.at[pl.ds(r0, RPS)],
                          out_hbm.at[c, pl.ds(r0, RPS)], zsem).wait()


_deg_kernel = functools.partial(
    pl.kernel,
    out_type=jax.ShapeDtypeStruct((NC, N_PAD, 8), jnp.float32),
    mesh=_sc_mesh(),
    compiler_params=pltpu.CompilerParams(use_tc_tiling_on_sc=False),
    scratch_types=[
        pltpu.VMEM((NPB, EB), jnp.int32),
        pltpu.VMEM((EB, 8), jnp.float32),
        pltpu.SemaphoreType.DMA,
        pltpu.SemaphoreType.DMA,
        pltpu.VMEM_SHARED((N_PAD, 8), jnp.float32),
    ],
)(_deg_body)


def _agg_body(y_hbm, src_hbm, dst_hbm, zeros_hbm, out_hbm, idx_s, idx_d, rows,
              gsem, ssem, acc_sh):
    c = lax.axis_index("c")
    s = lax.axis_index("s")
    wid = c * NS + s
    r0 = s * RPS
    ngrp, geb = idx_s.shape

    pltpu.async_copy(src_hbm.at[wid], idx_s, gsem.at[0])
    pltpu.async_copy(dst_hbm.at[wid], idx_d, gsem.at[1])
    pltpu.async_copy(zeros_hbm, acc_sh.at[pl.ds(r0, RPS)], ssem.at[0])
    pltpu.make_async_copy(zeros_hbm, acc_sh.at[pl.ds(r0, RPS)],
                          ssem.at[0]).wait()
    pltpu.make_async_copy(src_hbm.at[wid], idx_s, gsem.at[0]).wait()
    pltpu.make_async_copy(dst_hbm.at[wid], idx_d, gsem.at[1]).wait()
    plsc.subcore_barrier()

    nbuf = rows.shape[0]
    for p in range(nbuf):
        pltpu.async_copy(y_hbm.at[idx_s.at[p]], rows.at[p], gsem.at[p])

    def eloop(g, _):
        p = lax.rem(g, nbuf)
        pltpu.make_async_copy(y_hbm.at[pl.ds(0, geb)], rows.at[p],
                              gsem.at[p]).wait()
        pltpu.async_copy(rows.at[p], acc_sh.at[idx_d.at[g]], ssem.at[p],
                         add=True)
        pltpu.make_async_copy(y_hbm.at[pl.ds(0, geb)], rows.at[p],
                              ssem.at[p]).wait()

        @pl.when(g + nbuf < ngrp)
        def _next():
            pltpu.async_copy(y_hbm.at[idx_s.at[g + nbuf]], rows.at[p],
                             gsem.at[p])

        return _

    lax.fori_loop(0, ngrp, eloop, None)
    plsc.subcore_barrier()

    pltpu.async_copy(acc_sh.at[pl.ds(r0, RPS)], out_hbm.at[c, pl.ds(r0, RPS)],
                     gsem.at[0])
    pltpu.make_async_copy(acc_sh.at[pl.ds(r0, RPS)],
                          out_hbm.at[c, pl.ds(r0, RPS)], gsem.at[0]).wait()


_agg64 = functools.partial(
    pl.kernel,
    out_type=jax.ShapeDtypeStruct((NC, N_PAD, D), jnp.float32),
    mesh=_sc_mesh(),
    compiler_params=pltpu.CompilerParams(use_tc_tiling_on_sc=False),
    scratch_types=[
        pltpu.VMEM((NGRP, GEB), jnp.int32),
        pltpu.VMEM((NGRP, GEB), jnp.int32),
        pltpu.VMEM((4, GEB, D), jnp.float32),
        pltpu.SemaphoreType.DMA((4,)),
        pltpu.SemaphoreType.DMA((4,)),
        pltpu.VMEM_SHARED((N_PAD, D), jnp.float32),
    ],
)(_agg_body)

_agg64m = functools.partial(
    pl.kernel,
    out_type=jax.ShapeDtypeStruct((NC, N_PAD, D), jnp.float32),
    mesh=_sc_mesh(),
    compiler_params=pltpu.CompilerParams(use_tc_tiling_on_sc=False),
    scratch_types=[
        pltpu.VMEM((MNGRP, MGEB), jnp.int32),
        pltpu.VMEM((MNGRP, MGEB), jnp.int32),
        pltpu.VMEM((2, MGEB, D), jnp.float32),
        pltpu.SemaphoreType.DMA((2,)),
        pltpu.SemaphoreType.DMA((2,)),
        pltpu.VMEM_SHARED((N_PAD, D), jnp.float32),
    ],
)(_agg_body)


def _t1_body(x_ref, w_ref, deg_ref, y_ref, dinv_ref):
    deg = deg_ref[0, :, 0:1] + deg_ref[1, :, 0:1] + 1.0
    dinv = lax.rsqrt(deg)
    y_ref[...] = dinv * jnp.dot(x_ref[...], w_ref[...],
                                preferred_element_type=jnp.float32)
    dinv_ref[...] = jnp.broadcast_to(dinv, (N_PAD, 8))


def _t2_body(agg_ref, y_ref, dinv_ref, b_ref, wa_ref, wb_ref, y2_ref):
    dinv = dinv_ref[:, 0:1]
    h = jnp.maximum(dinv * (agg_ref[0] + agg_ref[1] + y_ref[...]) + b_ref[...],
                    0.0)
    y2_ref[0] = dinv * jnp.dot(h, wa_ref[...],
                               preferred_element_type=jnp.float32)
    y2_ref[1] = dinv * jnp.dot(h, wb_ref[...],
                               preferred_element_type=jnp.float32)


def _t3_body(agg_ref, y2_ref, dinv_ref, ba_ref, bb_ref, wa_ref, wb_ref,
             out_ref):
    dinv = dinv_ref[:, 0:1]
    ha = jnp.maximum(dinv * (agg_ref[0] + y2_ref[0]) + ba_ref[...], 0.0)
    hb = jnp.maximum(dinv * (agg_ref[1] + y2_ref[1]) + bb_ref[...], 0.0)
    out_ref[...] = dinv * (
        jnp.dot(ha, wa_ref[...], preferred_element_type=jnp.float32)
        + jnp.dot(hb, wb_ref[...], preferred_element_type=jnp.float32))


def _t4_body(agg_ref, y_ref, dinv_ref, b_ref, batch_ref, wlin_ref, blin_ref,
             out_ref):
    dinv = dinv_ref[:, 0:1]
    h = jnp.maximum(dinv * (agg_ref[0] + agg_ref[1] + y_ref[...]) + b_ref[...],
                    0.0)
    gids = lax.broadcasted_iota(jnp.int32, (N_GRAPHS, N_PAD), 0)
    onehot = (batch_ref[...] == gids).astype(jnp.float32)
    cnts = jnp.sum(onehot, axis=1, keepdims=True)
    sums = jnp.dot(onehot, h, preferred_element_type=jnp.float32)
    pooled = sums / jnp.maximum(cnts, 1.0)
    out_ref[...] = jnp.dot(pooled, wlin_ref[...],
                           preferred_element_type=jnp.float32) + blin_ref[...]


def kernel(x, edge_index, batch, W1, b1, W2, b2, W3, b3, Wlin, blin):
    n = x.shape[0]
    e = edge_index.shape[1]

    fill = n + (jnp.arange(E_PAD - e, dtype=jnp.int32) % (N_PAD - 8 - n))
    srcfull = jnp.concatenate([edge_index[0], fill])
    dstfull = jnp.concatenate([edge_index[1], fill])
    dst = dstfull.reshape(NW, NPB, EB)
    src4 = srcfull.reshape(NW, NGRP, GEB)
    dst4 = dst.reshape(NW, NGRP, GEB)

    src_sub = srcfull.reshape(NS, 2 * EPW)
    dst_sub = dstfull.reshape(NS, 2 * EPW)
    src_m = jnp.stack([src_sub, src_sub + N_PAD]).reshape(NW, MNGRP, MGEB)
    dst_m = jnp.stack([dst_sub, dst_sub]).reshape(NW, MNGRP, MGEB)
    x_p = jnp.pad(x, ((0, N_PAD - n), (0, 0)))
    batch_p = jnp.pad(batch, (0, N_PAD - n),
                      constant_values=N_GRAPHS).reshape(1, N_PAD)

    ones8 = jnp.ones((EB, 8), jnp.float32)
    zeros8 = jnp.zeros((RPS, 8), jnp.float32)
    zeros64 = jnp.zeros((RPS, D), jnp.float32)

    degraw = _deg_kernel(dst, ones8, zeros8)

    y1, dinv8 = pl.pallas_call(
        _t1_body,
        out_shape=[
            jax.ShapeDtypeStruct((N_PAD, 64), jnp.float32),
            jax.ShapeDtypeStruct((N_PAD, 8), jnp.float32),
        ],
    )(x_p, W1, degraw)

    agg1 = _agg64(y1, src4, dst4, zeros64)

    y2 = pl.pallas_call(
        _t2_body,
        out_shape=jax.ShapeDtypeStruct((2, N_PAD, 64), jnp.float32),
    )(agg1, y1, dinv8, b1.reshape(1, 64), W2[:, :64], W2[:, 64:])

    agg2 = _agg64m(y2.reshape(2 * N_PAD, 64), src_m, dst_m, zeros64)

    y3 = pl.pallas_call(
        _t3_body,
        out_shape=jax.ShapeDtypeStruct((N_PAD, 64), jnp.float32),
    )(agg2, y2, dinv8, b2[:64].reshape(1, 64),
      b2[64:].reshape(1, 64), W3[:64], W3[64:])

    agg3 = _agg64(y3, src4, dst4, zeros64)

    out = pl.pallas_call(
        _t4_body,
        out_shape=jax.ShapeDtypeStruct((N_GRAPHS, 1), jnp.float32),
    )(agg3, y3, dinv8, b3.reshape(1, 64), batch_p, Wlin, blin.reshape(1, 1))

    return out


# speedup vs baseline: 1.1045x; 1.0329x over previous
import functools

import jax
import jax.numpy as jnp
from jax import lax
from jax.experimental import pallas as pl
from jax.experimental.pallas import tpu as pltpu
from jax.experimental.pallas import tpu_sc as plsc

N_PAD = 10240
EB = 128
D = 64
NC, NS = 2, 16
NW = NC * NS
NPB = 80
EPW = NPB * EB
E_PAD = NW * EPW
RPS = N_PAD // NS
GEB = 128
NGRP = EPW // GEB
MGEB = 128
MNGRP = 2 * EPW // MGEB
NBUF = 8
N_GRAPHS = 64


def _sc_mesh():
    return plsc.VectorSubcoreMesh(core_axis_name="c", subcore_axis_name="s")


_DEG_LAG = 8


def _deg_body(dst_hbm, ones_hbm, zeros_hbm, out_hbm, idx_d, ones_v, zsem, sem,
              acc_sh):
    c = lax.axis_index("c")
    s = lax.axis_index("s")
    wid = c * NS + s
    r0 = s * RPS

    pltpu.async_copy(dst_hbm.at[wid], idx_d, sem)
    pltpu.async_copy(zeros_hbm, acc_sh.at[pl.ds(r0, RPS)], zsem)
    pltpu.sync_copy(ones_hbm, ones_v)
    pltpu.make_async_copy(zeros_hbm, acc_sh.at[pl.ds(r0, RPS)], zsem).wait()
    pltpu.make_async_copy(dst_hbm.at[wid], idx_d, sem).wait()
    plsc.subcore_barrier()

    def eloop(i, _):
        pltpu.async_copy(ones_v, acc_sh.at[idx_d.at[i]], sem, add=True)

        @pl.when(i >= _DEG_LAG)
        def _drain():
            pltpu.make_async_copy(ones_hbm, ones_v, sem).wait()

        return _

    lax.fori_loop(0, NPB, eloop, None)
    for _ in range(_DEG_LAG):
        pltpu.make_async_copy(ones_hbm, ones_v, sem).wait()
    plsc.subcore_barrier()

    pltpu.async_copy(acc_sh.at[pl.ds(r0, RPS)], out_hbm.at[c, pl.ds(r0, RPS)],
                     zsem)
    pltpu.make_async_copy(acc_sh.at[pl.ds(r0, RPS)],
                          out_hbm.at[c, pl.ds(r0, RPS)], zsem).wait()


_deg_kernel = functools.partial(
    pl.kernel,
    out_type=jax.ShapeDtypeStruct((NC, N_PAD, 8), jnp.float32),
    mesh=_sc_mesh(),
    compiler_params=pltpu.CompilerParams(use_tc_tiling_on_sc=False),
    scratch_types=[
        pltpu.VMEM((NPB, EB), jnp.int32),
        pltpu.VMEM((EB, 8), jnp.float32),
        pltpu.SemaphoreType.DMA,
        pltpu.SemaphoreType.DMA,
        pltpu.VMEM_SHARED((N_PAD, 8), jnp.float32),
    ],
)(_deg_body)


def _agg_body(y_hbm, src_hbm, dst_hbm, zeros_hbm, out_hbm, idx_s, idx_d, rows,
              gsem, ssem, acc_sh):
    c = lax.axis_index("c")
    s = lax.axis_index("s")
    wid = c * NS + s
    r0 = s * RPS
    ngrp, geb = idx_s.shape

    pltpu.async_copy(src_hbm.at[wid], idx_s, gsem.at[0])
    pltpu.async_copy(dst_hbm.at[wid], idx_d, gsem.at[1])
    pltpu.async_copy(zeros_hbm, acc_sh.at[pl.ds(r0, RPS)], ssem.at[0])
    pltpu.make_async_copy(zeros_hbm, acc_sh.at[pl.ds(r0, RPS)],
                          ssem.at[0]).wait()
    pltpu.make_async_copy(src_hbm.at[wid], idx_s, gsem.at[0]).wait()
    pltpu.make_async_copy(dst_hbm.at[wid], idx_d, gsem.at[1]).wait()
    plsc.subcore_barrier()

    nbuf = rows.shape[0]
    for p in range(nbuf):
        pltpu.async_copy(y_hbm.at[idx_s.at[p]], rows.at[p], gsem.at[p])

    def eloop(g, _):
        p = lax.rem(g, nbuf)
        pltpu.make_async_copy(y_hbm.at[pl.ds(0, geb)], rows.at[p],
                              gsem.at[p]).wait()
        pltpu.async_copy(rows.at[p], acc_sh.at[idx_d.at[g]], ssem.at[p],
                         add=True)
        pltpu.make_async_copy(y_hbm.at[pl.ds(0, geb)], rows.at[p],
                              ssem.at[p]).wait()

        @pl.when(g + nbuf < ngrp)
        def _next():
            pltpu.async_copy(y_hbm.at[idx_s.at[g + nbuf]], rows.at[p],
                             gsem.at[p])

        return _

    lax.fori_loop(0, ngrp, eloop, None)
    plsc.subcore_barrier()

    pltpu.async_copy(acc_sh.at[pl.ds(r0, RPS)], out_hbm.at[c, pl.ds(r0, RPS)],
                     gsem.at[0])
    pltpu.make_async_copy(acc_sh.at[pl.ds(r0, RPS)],
                          out_hbm.at[c, pl.ds(r0, RPS)], gsem.at[0]).wait()


_agg64 = functools.partial(
    pl.kernel,
    out_type=jax.ShapeDtypeStruct((NC, N_PAD, D), jnp.float32),
    mesh=_sc_mesh(),
    compiler_params=pltpu.CompilerParams(use_tc_tiling_on_sc=False),
    scratch_types=[
        pltpu.VMEM((NGRP, GEB), jnp.int32),
        pltpu.VMEM((NGRP, GEB), jnp.int32),
        pltpu.VMEM((8, GEB, D), jnp.float32),
        pltpu.SemaphoreType.DMA((8,)),
        pltpu.SemaphoreType.DMA((8,)),
        pltpu.VMEM_SHARED((N_PAD, D), jnp.float32),
    ],
)(_agg_body)

_agg64m = functools.partial(
    pl.kernel,
    out_type=jax.ShapeDtypeStruct((NC, N_PAD, D), jnp.float32),
    mesh=_sc_mesh(),
    compiler_params=pltpu.CompilerParams(use_tc_tiling_on_sc=False),
    scratch_types=[
        pltpu.VMEM((MNGRP, MGEB), jnp.int32),
        pltpu.VMEM((MNGRP, MGEB), jnp.int32),
        pltpu.VMEM((4, MGEB, D), jnp.float32),
        pltpu.SemaphoreType.DMA((4,)),
        pltpu.SemaphoreType.DMA((4,)),
        pltpu.VMEM_SHARED((N_PAD, D), jnp.float32),
    ],
)(_agg_body)


def _t1_body(x_ref, w_ref, deg_ref, y_ref, dinv_ref):
    deg = deg_ref[0, :, 0:1] + deg_ref[1, :, 0:1] + 1.0
    dinv = lax.rsqrt(deg)
    y_ref[...] = dinv * jnp.dot(x_ref[...], w_ref[...],
                                preferred_element_type=jnp.float32)
    dinv_ref[...] = jnp.broadcast_to(dinv, (N_PAD, 8))


def _t2_body(agg_ref, y_ref, dinv_ref, b_ref, wa_ref, wb_ref, y2_ref):
    dinv = dinv_ref[:, 0:1]
    h = jnp.maximum(dinv * (agg_ref[0] + agg_ref[1] + y_ref[...]) + b_ref[...],
                    0.0)
    y2_ref[0] = dinv * jnp.dot(h, wa_ref[...],
                               preferred_element_type=jnp.float32)
    y2_ref[1] = dinv * jnp.dot(h, wb_ref[...],
                               preferred_element_type=jnp.float32)


def _t3_body(agg_ref, y2_ref, dinv_ref, ba_ref, bb_ref, wa_ref, wb_ref,
             out_ref):
    dinv = dinv_ref[:, 0:1]
    ha = jnp.maximum(dinv * (agg_ref[0] + y2_ref[0]) + ba_ref[...], 0.0)
    hb = jnp.maximum(dinv * (agg_ref[1] + y2_ref[1]) + bb_ref[...], 0.0)
    out_ref[...] = dinv * (
        jnp.dot(ha, wa_ref[...], preferred_element_type=jnp.float32)
        + jnp.dot(hb, wb_ref[...], preferred_element_type=jnp.float32))


def _t4_body(agg_ref, y_ref, dinv_ref, b_ref, batch_ref, wlin_ref, blin_ref,
             out_ref):
    dinv = dinv_ref[:, 0:1]
    h = jnp.maximum(dinv * (agg_ref[0] + agg_ref[1] + y_ref[...]) + b_ref[...],
                    0.0)
    gids = lax.broadcasted_iota(jnp.int32, (N_GRAPHS, N_PAD), 0)
    onehot = (batch_ref[...] == gids).astype(jnp.float32)
    cnts = jnp.sum(onehot, axis=1, keepdims=True)
    sums = jnp.dot(onehot, h, preferred_element_type=jnp.float32)
    pooled = sums / jnp.maximum(cnts, 1.0)
    out_ref[...] = jnp.dot(pooled, wlin_ref[...],
                           preferred_element_type=jnp.float32) + blin_ref[...]


def kernel(x, edge_index, batch, W1, b1, W2, b2, W3, b3, Wlin, blin):
    n = x.shape[0]
    e = edge_index.shape[1]

    fill = n + (jnp.arange(E_PAD - e, dtype=jnp.int32) % (N_PAD - 8 - n))
    srcfull = jnp.concatenate([edge_index[0], fill])
    dstfull = jnp.concatenate([edge_index[1], fill])
    dst = dstfull.reshape(NW, NPB, EB)
    src4 = srcfull.reshape(NW, NGRP, GEB)
    dst4 = dst.reshape(NW, NGRP, GEB)

    src_sub = srcfull.reshape(NS, 2 * EPW)
    dst_sub = dstfull.reshape(NS, 2 * EPW)
    src_m = jnp.stack([src_sub, src_sub + N_PAD]).reshape(NW, MNGRP, MGEB)
    dst_m = jnp.stack([dst_sub, dst_sub]).reshape(NW, MNGRP, MGEB)
    x_p = jnp.pad(x, ((0, N_PAD - n), (0, 0)))
    batch_p = jnp.pad(batch, (0, N_PAD - n),
                      constant_values=N_GRAPHS).reshape(1, N_PAD)

    ones8 = jnp.ones((EB, 8), jnp.float32)
    zeros8 = jnp.zeros((RPS, 8), jnp.float32)
    zeros64 = jnp.zeros((RPS, D), jnp.float32)

    degraw = _deg_kernel(dst, ones8, zeros8)

    y1, dinv8 = pl.pallas_call(
        _t1_body,
        out_shape=[
            jax.ShapeDtypeStruct((N_PAD, 64), jnp.float32),
            jax.ShapeDtypeStruct((N_PAD, 8), jnp.float32),
        ],
    )(x_p, W1, degraw)

    agg1 = _agg64(y1, src4, dst4, zeros64)

    y2 = pl.pallas_call(
        _t2_body,
        out_shape=jax.ShapeDtypeStruct((2, N_PAD, 64), jnp.float32),
    )(agg1, y1, dinv8, b1.reshape(1, 64), W2[:, :64], W2[:, 64:])

    agg2 = _agg64m(y2.reshape(2 * N_PAD, 64), src_m, dst_m, zeros64)

    y3 = pl.pallas_call(
        _t3_body,
        out_shape=jax.ShapeDtypeStruct((N_PAD, 64), jnp.float32),
    )(agg2, y2, dinv8, b2[:64].reshape(1, 64),
      b2[64:].reshape(1, 64), W3[:64], W3[64:])

    agg3 = _agg64(y3, src4, dst4, zeros64)

    out = pl.pallas_call(
        _t4_body,
        out_shape=jax.ShapeDtypeStruct((N_GRAPHS, 1), jnp.float32),
    )(agg3, y3, dinv8, b3.reshape(1, 64), batch_p, Wlin, blin.reshape(1, 1))

    return out
